# baseline (speedup 1.0000x reference)
import functools

import jax
import jax.numpy as jnp
from jax import lax
from jax.experimental import pallas as pl
from jax.experimental.pallas import tpu as pltpu
from jax.experimental.pallas import tpu_sc as plsc


def _encoding_table(length: int, size: int) -> jax.Array:
    depth = size // 2
    positions = jnp.arange(length, dtype=jnp.float32)[:, None]
    depths = jnp.arange(depth, dtype=jnp.float32)[None, :] / depth
    angle_rates = 1.0 / (10000.0 ** depths)
    angle_rads = positions * angle_rates
    enc = jnp.concatenate([jnp.sin(angle_rads), jnp.cos(angle_rads)], axis=-1)
    return jnp.pad(enc, ((0, 0), (0, size)))


@functools.partial(jax.jit, static_argnums=(2, 3))
def _sc_gather(table, idx, b, size):
    info = plsc.get_sparse_core_info()
    nc, ns = info.num_cores, info.num_subcores
    nw = nc * ns
    b_per_w = b // nw
    chunk = next(c for c in range(256, 0, -8)
                 if b_per_w % c == 0 and (b_per_w // c) % 2 == 0)
    n_chunks = b_per_w // chunk
    n_rounds = n_chunks // 2
    rpc = chunk * size // 256
    rows_total = b * size // 256

    mesh = plsc.VectorSubcoreMesh(core_axis_name="c", subcore_axis_name="s")

    @functools.partial(
        pl.kernel,
        mesh=mesh,
        out_type=jax.ShapeDtypeStruct((rows_total, 256), jnp.float32),
        compiler_params=pltpu.CompilerParams(use_tc_tiling_on_sc=True),
        scratch_types=[
            pltpu.VMEM((chunk,), jnp.int32),
            pltpu.VMEM((chunk,), jnp.int32),
            pltpu.VMEM((chunk, 2 * size), jnp.float32),
            pltpu.VMEM((chunk, 2 * size), jnp.float32),
            pltpu.VMEM((rpc, 256), jnp.float32),
            pltpu.VMEM((rpc, 256), jnp.float32),
            pltpu.SemaphoreType.DMA,
            pltpu.SemaphoreType.DMA,
            pltpu.SemaphoreType.DMA,
            pltpu.SemaphoreType.DMA,
            pltpu.SemaphoreType.DMA,
            pltpu.SemaphoreType.DMA,
        ],
    )
    def k(table_hbm, idx_hbm, out_hbm, i0, i1, g0, g1, p0, p1,
          si0, si1, sg0, sg1, so0, so1):
        idx_v, raw_v, pak_v = (i0, i1), (g0, g1), (p0, p1)
        sidx, sgat, sout = (si0, si1), (sg0, sg1), (so0, so1)
        wid = lax.axis_index("s") * nc + lax.axis_index("c")
        base = wid * b_per_w
        rbase = wid * (b_per_w * size // 256)

        def start_idx(slot, i):
            pltpu.async_copy(
                idx_hbm.at[pl.ds(base + i * chunk, chunk)], idx_v[slot],
                sidx[slot])

        def wait_idx(slot, i):
            pltpu.make_async_copy(
                idx_hbm.at[pl.ds(base + i * chunk, chunk)], idx_v[slot],
                sidx[slot]).wait()

        def start_out(slot, i):
            pltpu.async_copy(
                pak_v[slot], out_hbm.at[pl.ds(rbase + i * rpc, rpc)],
                sout[slot])

        def wait_out(slot, i):
            pltpu.make_async_copy(
                pak_v[slot], out_hbm.at[pl.ds(rbase + i * rpc, rpc)],
                sout[slot]).wait()

        def gather(slot):
            pltpu.async_copy(table_hbm.at[idx_v[slot]], raw_v[slot],
                             sgat[slot]).wait()

        def repack(slot):
            src, dst = raw_v[slot], pak_v[slot]

            def body(r, carry):
                q = r * 4
                for dcoord in range(4):
                    for j in range(4):
                        dst[r, pl.ds(dcoord * 64 + j * 16, 16)] = (
                            src[q + dcoord, pl.ds(j * 16, 16)])
                return carry

            lax.fori_loop(0, rpc, body, 0, unroll=2)

        start_idx(0, 0)
        start_idx(1, 1)
        for slot in (0, 1):
            wait_idx(slot, slot)
            gather(slot)
            start_idx(slot, slot + 2)
            repack(slot)
            start_out(slot, slot)

        def round_body(r, carry):
            for slot in (0, 1):
                i = 2 * r + slot
                wait_idx(slot, i)
                gather(slot)
                start_idx(slot, i + 2)
                wait_out(slot, i - 2)
                repack(slot)
                start_out(slot, i)
            return carry

        lax.fori_loop(1, n_rounds - 1, round_body, 0)

        for slot in (0, 1):
            i = 2 * (n_rounds - 1) + slot
            wait_idx(slot, i)
            gather(slot)
            wait_out(slot, i - 2)
            repack(slot)
            start_out(slot, i)
        for slot in (0, 1):
            wait_out(slot, 2 * (n_rounds - 1) + slot)

    return k(table, idx)


def kernel(boxes):
    n, w, d = boxes.shape
    size = 256 // d
    table = _encoding_table(2048, size)
    idx = jnp.transpose(jnp.round(boxes).astype(jnp.int32), (1, 0, 2)).reshape(-1)
    out = _sc_gather(table, idx, idx.shape[0], size)
    return jnp.transpose(out.reshape(w, n, d * size), (1, 0, 2))


# speedup vs baseline: 1.1269x; 1.1269x over previous
import functools

import jax
import jax.numpy as jnp
from jax import lax
from jax.experimental import pallas as pl
from jax.experimental.pallas import tpu as pltpu
from jax.experimental.pallas import tpu_sc as plsc


def _encoding_table(length: int, size: int) -> jax.Array:
    depth = size // 2
    positions = jnp.arange(length, dtype=jnp.float32)[:, None]
    depths = jnp.arange(depth, dtype=jnp.float32)[None, :] / depth
    angle_rates = 1.0 / (10000.0 ** depths)
    angle_rads = positions * angle_rates
    enc = jnp.concatenate([jnp.sin(angle_rads), jnp.cos(angle_rads)], axis=-1)
    return jnp.pad(enc, ((0, 0), (0, size)))


@functools.partial(jax.jit, static_argnums=(2, 3))
def _sc_gather(table, idx, b, size):
    info = plsc.get_sparse_core_info()
    nc, ns = info.num_cores, info.num_subcores
    nw = nc * ns
    b_per_w = b // nw
    chunk = next(c for c in range(256, 0, -8)
                 if b_per_w % c == 0 and (b_per_w // c) % 2 == 0)
    n_chunks = b_per_w // chunk
    n_rounds = n_chunks // 2
    rpc = chunk * size // 256
    rows_total = b * size // 256

    mesh = plsc.VectorSubcoreMesh(core_axis_name="c", subcore_axis_name="s")

    @functools.partial(
        pl.kernel,
        mesh=mesh,
        out_type=jax.ShapeDtypeStruct((rows_total, 256), jnp.float32),
        compiler_params=pltpu.CompilerParams(use_tc_tiling_on_sc=True),
        scratch_types=[
            pltpu.VMEM((chunk,), jnp.int32),
            pltpu.VMEM((chunk,), jnp.int32),
            pltpu.VMEM((chunk, 2 * size), jnp.float32),
            pltpu.VMEM((chunk, 2 * size), jnp.float32),
            pltpu.VMEM((rpc, 256), jnp.float32),
            pltpu.VMEM((rpc, 256), jnp.float32),
            pltpu.SemaphoreType.DMA,
            pltpu.SemaphoreType.DMA,
            pltpu.SemaphoreType.DMA,
            pltpu.SemaphoreType.DMA,
            pltpu.SemaphoreType.DMA,
            pltpu.SemaphoreType.DMA,
        ],
    )
    def k(table_hbm, idx_hbm, out_hbm, i0, i1, g0, g1, p0, p1,
          si0, si1, sg0, sg1, so0, so1):
        idx_v, raw_v, pak_v = (i0, i1), (g0, g1), (p0, p1)
        sidx, sgat, sout = (si0, si1), (sg0, sg1), (so0, so1)
        wid = lax.axis_index("s") * nc + lax.axis_index("c")
        base = wid * b_per_w
        rbase = wid * (b_per_w * size // 256)

        def start_idx(slot, i):
            pltpu.async_copy(
                idx_hbm.at[pl.ds(base + i * chunk, chunk)], idx_v[slot],
                sidx[slot])

        def wait_idx(slot, i):
            pltpu.make_async_copy(
                idx_hbm.at[pl.ds(base + i * chunk, chunk)], idx_v[slot],
                sidx[slot]).wait()

        def start_out(slot, i):
            pltpu.async_copy(
                pak_v[slot], out_hbm.at[pl.ds(rbase + i * rpc, rpc)],
                sout[slot])

        def wait_out(slot, i):
            pltpu.make_async_copy(
                pak_v[slot], out_hbm.at[pl.ds(rbase + i * rpc, rpc)],
                sout[slot]).wait()

        def start_gather(slot):
            pltpu.async_copy(table_hbm.at[idx_v[slot]], raw_v[slot],
                             sgat[slot])

        def wait_gather(slot):
            pltpu.make_async_copy(table_hbm.at[idx_v[slot]], raw_v[slot],
                                  sgat[slot]).wait()

        def repack(slot):
            src, dst = raw_v[slot], pak_v[slot]

            def body(r, carry):
                q = r * 4
                for dcoord in range(4):
                    for j in range(4):
                        dst[r, pl.ds(dcoord * 64 + j * 16, 16)] = (
                            src[q + dcoord, pl.ds(j * 16, 16)])
                return carry

            lax.fori_loop(0, rpc, body, 0, unroll=8)

        start_idx(0, 0)
        start_idx(1, 1)
        wait_idx(0, 0)
        start_gather(0)
        wait_idx(1, 1)
        start_gather(1)
        for slot in (0, 1):
            wait_gather(slot)
            start_idx(slot, slot + 2)
            repack(slot)
            start_out(slot, slot)

        def round_body(r, carry):
            i0 = 2 * r
            wait_idx(0, i0)
            start_gather(0)
            wait_idx(1, i0 + 1)
            start_gather(1)
            for slot in (0, 1):
                i = i0 + slot
                wait_gather(slot)
                start_idx(slot, i + 2)
                wait_out(slot, i - 2)
                repack(slot)
                start_out(slot, i)
            return carry

        lax.fori_loop(1, n_rounds - 1, round_body, 0)

        i0 = 2 * (n_rounds - 1)
        wait_idx(0, i0)
        start_gather(0)
        wait_idx(1, i0 + 1)
        start_gather(1)
        for slot in (0, 1):
            wait_gather(slot)
            wait_out(slot, i0 + slot - 2)
            repack(slot)
            start_out(slot, i0 + slot)
        for slot in (0, 1):
            wait_out(slot, i0 + slot)

    return k(table, idx)


def kernel(boxes):
    n, w, d = boxes.shape
    size = 256 // d
    table = _encoding_table(2048, size)
    idx = jnp.transpose(jnp.round(boxes).astype(jnp.int32), (1, 0, 2)).reshape(-1)
    out = _sc_gather(table, idx, idx.shape[0], size)
    return jnp.transpose(out.reshape(w, n, d * size), (1, 0, 2))


# speedup vs baseline: 1.6221x; 1.4394x over previous
import functools

import jax
import jax.numpy as jnp
from jax import lax
from jax.experimental import pallas as pl
from jax.experimental.pallas import tpu as pltpu
from jax.experimental.pallas import tpu_sc as plsc


def _encoding_table(length: int, size: int) -> jax.Array:
    depth = size // 2
    positions = jnp.arange(length, dtype=jnp.float32)[:, None]
    depths = jnp.arange(depth, dtype=jnp.float32)[None, :] / depth
    angle_rates = 1.0 / (10000.0 ** depths)
    angle_rads = positions * angle_rates
    return jnp.concatenate([jnp.sin(angle_rads), jnp.cos(angle_rads)], axis=-1)


@functools.partial(jax.jit, static_argnums=(2,))
def _sc_gather(table, idx4, size):
    w, nb, d, nl = idx4.shape
    n = nb * nl
    rows_total = w * n
    info = plsc.get_sparse_core_info()
    nc, ns = info.num_cores, info.num_subcores
    nw = nc * ns
    rows_per_w = rows_total // nw
    chunk_rows = nl
    n_chunks = rows_per_w // chunk_rows
    n_rounds = n_chunks // 2

    mesh = plsc.VectorSubcoreMesh(core_axis_name="c", subcore_axis_name="s")

    @functools.partial(
        pl.kernel,
        mesh=mesh,
        out_type=jax.ShapeDtypeStruct((rows_total, d * size), jnp.float32),
        compiler_params=pltpu.CompilerParams(use_tc_tiling_on_sc=False),
        scratch_types=[
            pltpu.VMEM((d, nl), jnp.int32),
            pltpu.VMEM((d, nl), jnp.int32),
            pltpu.VMEM((d, nl, size), jnp.float32),
            pltpu.VMEM((d, nl, size), jnp.float32),
            pltpu.SemaphoreType.DMA,
            pltpu.SemaphoreType.DMA,
            pltpu.SemaphoreType.DMA,
            pltpu.SemaphoreType.DMA,
            pltpu.SemaphoreType.DMA,
            pltpu.SemaphoreType.DMA,
        ],
    )
    def k(table_hbm, idx_hbm, out_hbm, i0, i1, g0, g1,
          si0, si1, sg0, sg1, so0, so1):
        idx_v, raw_v = (i0, i1), (g0, g1)
        sidx, sgat, sout = (si0, si1), (sg0, sg1), (so0, so1)
        wid = lax.axis_index("s") * nc + lax.axis_index("c")
        rbase = wid * rows_per_w

        def start_idx(slot, i):
            r = rbase + i * chunk_rows
            pltpu.async_copy(idx_hbm.at[r // n, (r % n) // nl], idx_v[slot],
                             sidx[slot])

        def wait_idx(slot, i):
            r = rbase + i * chunk_rows
            pltpu.make_async_copy(idx_hbm.at[r // n, (r % n) // nl],
                                  idx_v[slot], sidx[slot]).wait()

        def start_gathers(slot):
            for dc in range(d):
                pltpu.async_copy(table_hbm.at[idx_v[slot].at[dc]],
                                 raw_v[slot].at[dc], sgat[slot])

        def drain_gathers(slot):
            for dc in range(d):
                pltpu.make_async_copy(table_hbm.at[idx_v[slot].at[dc]],
                                      raw_v[slot].at[dc], sgat[slot]).wait()

        def start_out(slot, i):
            r = rbase + i * chunk_rows
            for dc in range(d):
                pltpu.async_copy(
                    raw_v[slot].at[dc],
                    out_hbm.at[pl.ds(r, chunk_rows),
                               pl.ds(dc * size, size)],
                    sout[slot])

        def wait_out(slot, i):
            r = rbase + i * chunk_rows
            for dc in range(d):
                pltpu.make_async_copy(
                    raw_v[slot].at[dc],
                    out_hbm.at[pl.ds(r, chunk_rows),
                               pl.ds(dc * size, size)],
                    sout[slot]).wait()

        start_idx(0, 0)
        start_idx(1, 1)
        wait_idx(0, 0)
        start_gathers(0)
        wait_idx(1, 1)
        start_gathers(1)
        for slot in (0, 1):
            drain_gathers(slot)
            start_idx(slot, slot + 2)
            start_out(slot, slot)

        def round_body(r, carry):
            i0b = 2 * r
            wait_idx(0, i0b)
            wait_out(0, i0b - 2)
            start_gathers(0)
            wait_idx(1, i0b + 1)
            wait_out(1, i0b - 1)
            start_gathers(1)
            for slot in (0, 1):
                drain_gathers(slot)
                start_idx(slot, i0b + slot + 2)
                start_out(slot, i0b + slot)
            return carry

        lax.fori_loop(1, n_rounds - 1, round_body, 0)

        i0b = 2 * (n_rounds - 1)
        wait_idx(0, i0b)
        wait_out(0, i0b - 2)
        start_gathers(0)
        wait_idx(1, i0b + 1)
        wait_out(1, i0b - 1)
        start_gathers(1)
        for slot in (0, 1):
            drain_gathers(slot)
            start_out(slot, i0b + slot)
        for slot in (0, 1):
            wait_out(slot, i0b + slot)

    return k(table, idx4)


def kernel(boxes):
    n, w, d = boxes.shape
    size = 256 // d
    nl = 128
    table = _encoding_table(2048, size)
    q = jnp.round(boxes).astype(jnp.int32)
    idx4 = (q.transpose(1, 0, 2)
             .reshape(w, n // nl, nl, d)
             .transpose(0, 1, 3, 2))
    out = _sc_gather(table, idx4, size)
    return jnp.transpose(out.reshape(w, n, d * size), (1, 0, 2))


# speedup vs baseline: 2.1144x; 1.3035x over previous
import functools

import jax
import jax.numpy as jnp
from jax import lax
from jax.experimental import pallas as pl
from jax.experimental.pallas import tpu as pltpu
from jax.experimental.pallas import tpu_sc as plsc


def _encoding_table(length: int, size: int) -> jax.Array:
    depth = size // 2
    positions = jnp.arange(length, dtype=jnp.float32)[:, None]
    depths = jnp.arange(depth, dtype=jnp.float32)[None, :] / depth
    angle_rates = 1.0 / (10000.0 ** depths)
    angle_rads = positions * angle_rates
    return jnp.concatenate([jnp.sin(angle_rads), jnp.cos(angle_rads)], axis=-1)


@functools.partial(jax.jit, static_argnums=(2,))
def _sc_gather(table, idx4, size):
    w, nb, d, nl = idx4.shape
    n = nb * nl
    rows_total = w * n
    info = plsc.get_sparse_core_info()
    nc, ns = info.num_cores, info.num_subcores
    nw = nc * ns
    rows_per_w = rows_total // nw
    chunk_rows = nl
    n_chunks = rows_per_w // chunk_rows
    n_rounds = n_chunks // 2

    mesh = plsc.VectorSubcoreMesh(core_axis_name="c", subcore_axis_name="s")

    @functools.partial(
        pl.kernel,
        mesh=mesh,
        out_type=jax.ShapeDtypeStruct((rows_total, d * size), jnp.float32),
        compiler_params=pltpu.CompilerParams(use_tc_tiling_on_sc=False),
        scratch_types=[
            pltpu.VMEM((d, nl), jnp.int32),
            pltpu.VMEM((d, nl), jnp.int32),
            pltpu.VMEM((d, nl, size), jnp.float32),
            pltpu.VMEM((d, nl, size), jnp.float32),
            pltpu.VMEM_SHARED((2048, 64), jnp.float32),
            pltpu.SemaphoreType.DMA,
            pltpu.SemaphoreType.DMA,
            pltpu.SemaphoreType.DMA,
            pltpu.SemaphoreType.DMA,
            pltpu.SemaphoreType.DMA,
            pltpu.SemaphoreType.DMA,
        ],
    )
    def k(table_hbm, idx_hbm, out_hbm, i0, i1, g0, g1, table_sp,
          si0, si1, sg0, sg1, so0, so1):
        idx_v, raw_v = (i0, i1), (g0, g1)
        sidx, sgat, sout = (si0, si1), (sg0, sg1), (so0, so1)
        wid = lax.axis_index("s") * nc + lax.axis_index("c")
        rbase = wid * rows_per_w

        @pl.when(lax.axis_index("s") == 0)
        def _stage():
            pltpu.sync_copy(table_hbm, table_sp)

        plsc.subcore_barrier()

        def start_idx(slot, i):
            r = rbase + i * chunk_rows
            pltpu.async_copy(idx_hbm.at[r // n, (r % n) // nl], idx_v[slot],
                             sidx[slot])

        def wait_idx(slot, i):
            r = rbase + i * chunk_rows
            pltpu.make_async_copy(idx_hbm.at[r // n, (r % n) // nl],
                                  idx_v[slot], sidx[slot]).wait()

        def start_gathers(slot):
            for dc in range(d):
                pltpu.async_copy(table_sp.at[idx_v[slot].at[dc]],
                                 raw_v[slot].at[dc], sgat[slot])

        def drain_gathers(slot):
            for dc in range(d):
                pltpu.make_async_copy(table_sp.at[idx_v[slot].at[dc]],
                                      raw_v[slot].at[dc], sgat[slot]).wait()

        def start_out(slot, i):
            r = rbase + i * chunk_rows
            for dc in range(d):
                pltpu.async_copy(
                    raw_v[slot].at[dc],
                    out_hbm.at[pl.ds(r, chunk_rows),
                               pl.ds(dc * size, size)],
                    sout[slot])

        def wait_out(slot, i):
            r = rbase + i * chunk_rows
            for dc in range(d):
                pltpu.make_async_copy(
                    raw_v[slot].at[dc],
                    out_hbm.at[pl.ds(r, chunk_rows),
                               pl.ds(dc * size, size)],
                    sout[slot]).wait()

        start_idx(0, 0)
        start_idx(1, 1)
        wait_idx(0, 0)
        start_gathers(0)
        wait_idx(1, 1)
        start_gathers(1)
        for slot in (0, 1):
            drain_gathers(slot)
            start_idx(slot, slot + 2)
            start_out(slot, slot)

        def round_body(r, carry):
            i0b = 2 * r
            wait_idx(0, i0b)
            wait_out(0, i0b - 2)
            start_gathers(0)
            wait_idx(1, i0b + 1)
            wait_out(1, i0b - 1)
            start_gathers(1)
            for slot in (0, 1):
                drain_gathers(slot)
                start_idx(slot, i0b + slot + 2)
                start_out(slot, i0b + slot)
            return carry

        lax.fori_loop(1, n_rounds - 1, round_body, 0)

        i0b = 2 * (n_rounds - 1)
        wait_idx(0, i0b)
        wait_out(0, i0b - 2)
        start_gathers(0)
        wait_idx(1, i0b + 1)
        wait_out(1, i0b - 1)
        start_gathers(1)
        for slot in (0, 1):
            drain_gathers(slot)
            start_out(slot, i0b + slot)
        for slot in (0, 1):
            wait_out(slot, i0b + slot)

    return k(table, idx4)


def kernel(boxes):
    n, w, d = boxes.shape
    size = 256 // d
    nl = 128
    table = _encoding_table(2048, size)
    q = jnp.round(boxes).astype(jnp.int32)
    idx4 = (q.transpose(1, 0, 2)
             .reshape(w, n // nl, nl, d)
             .transpose(0, 1, 3, 2))
    out = _sc_gather(table, idx4, size)
    return jnp.transpose(out.reshape(w, n, d * size), (1, 0, 2))
